# baseline (speedup 1.0000x reference)
import functools

import jax
import jax.numpy as jnp
from jax import lax
from jax.experimental import pallas as pl
from jax.experimental.pallas import tpu as pltpu
from jax.experimental.pallas import tpu_sc as plsc

_NC, _NS, _NL = 2, 16, 16
_NW = _NC * _NS
_G = 4


def kernel(x, emb_table):
    B, S = x.shape
    V, D = emb_table.shape
    rows = S * _G // _NW
    planes = B // _G

    @functools.partial(
        pl.kernel,
        mesh=plsc.VectorSubcoreMesh(core_axis_name="c", subcore_axis_name="s"),
        out_type=jax.ShapeDtypeStruct((B, S, D), jnp.float32),
        scratch_types=[
            pltpu.VMEM((rows, D), jnp.float32),
            pltpu.VMEM((rows, D), jnp.float32),
            pltpu.VMEM((rows, D), jnp.float32),
            pltpu.SemaphoreType.DMA,
        ],
    )
    def k(x_hbm, tab_hbm, out_hbm, xv, tv, yv, sem):
        wid = lax.axis_index("s") * _NC + lax.axis_index("c")
        group = wid // _G
        member = wid % _G
        s0 = group * rows

        @pl.when(s0 < V)
        def _():
            pltpu.sync_copy(x_hbm.at[pl.ds(s0, rows)], xv)
            pltpu.sync_copy(tab_hbm.at[pl.ds(s0, rows)], tv)
            for r in range(rows):
                for j in range(D // _NL):
                    sl = pl.ds(j * _NL, _NL)
                    yv[r, sl] = xv[r, sl] + tv[r, sl]

        @pl.when(s0 >= V)
        def _():
            nan16 = jnp.full((_NL,), jnp.nan, dtype=jnp.float32)
            for r in range(rows):
                for j in range(D // _NL):
                    yv[r, pl.ds(j * _NL, _NL)] = nan16

        b0 = member * planes
        unroll = 8

        def fire(g, carry):
            for u in range(unroll):
                pltpu.async_copy(yv, out_hbm.at[b0 + g * unroll + u,
                                                pl.ds(s0, rows)], sem)
            return carry

        lax.fori_loop(0, planes // unroll, fire, 0)
        pltpu.make_async_copy(
            out_hbm.at[pl.ds(b0, planes), pl.ds(s0, rows)],
            out_hbm.at[pl.ds(b0, planes), pl.ds(s0, rows)], sem).wait()

    return k(x, emb_table)


# speedup vs baseline: 1.1022x; 1.1022x over previous
import functools

import jax
import jax.numpy as jnp
from jax import lax
from jax.experimental import pallas as pl
from jax.experimental.pallas import tpu as pltpu
from jax.experimental.pallas import tpu_sc as plsc

_NC, _NS, _NL = 2, 16, 16
_NW = _NC * _NS
_G = 2


def kernel(x, emb_table):
    B, S = x.shape
    V, D = emb_table.shape
    rows = S * _G // _NW
    planes = B // _G

    @functools.partial(
        pl.kernel,
        mesh=plsc.VectorSubcoreMesh(core_axis_name="c", subcore_axis_name="s"),
        out_type=jax.ShapeDtypeStruct((B, S, D), jnp.float32),
        scratch_types=[
            pltpu.VMEM((rows, D), jnp.float32),
            pltpu.VMEM((rows, D), jnp.float32),
            pltpu.VMEM((rows, D), jnp.float32),
            pltpu.SemaphoreType.DMA,
        ],
    )
    def k(x_hbm, tab_hbm, out_hbm, xv, tv, yv, sem):
        wid = lax.axis_index("s") * _NC + lax.axis_index("c")
        group = wid // _G
        member = wid % _G
        s0 = group * rows

        @pl.when(s0 < V)
        def _():
            pltpu.sync_copy(x_hbm.at[pl.ds(s0, rows)], xv)
            pltpu.sync_copy(tab_hbm.at[pl.ds(s0, rows)], tv)
            for r in range(rows):
                for j in range(D // _NL):
                    sl = pl.ds(j * _NL, _NL)
                    yv[r, sl] = xv[r, sl] + tv[r, sl]

        @pl.when(s0 >= V)
        def _():
            nan16 = jnp.full((_NL,), jnp.nan, dtype=jnp.float32)
            for r in range(rows):
                for j in range(D // _NL):
                    yv[r, pl.ds(j * _NL, _NL)] = nan16

        b0 = member * planes
        unroll = 8

        def fire(g, carry):
            for u in range(unroll):
                pltpu.async_copy(yv, out_hbm.at[b0 + g * unroll + u,
                                                pl.ds(s0, rows)], sem)
            return carry

        lax.fori_loop(0, planes // unroll, fire, 0)
        pltpu.make_async_copy(
            out_hbm.at[pl.ds(b0, planes), pl.ds(s0, rows)],
            out_hbm.at[pl.ds(b0, planes), pl.ds(s0, rows)], sem).wait()

    return k(x, emb_table)


# speedup vs baseline: 1.1560x; 1.0488x over previous
import functools

import jax
import jax.numpy as jnp
from jax import lax
from jax.experimental import pallas as pl
from jax.experimental.pallas import tpu as pltpu
from jax.experimental.pallas import tpu_sc as plsc

_NC, _NS, _NL = 2, 16, 16
_NW = _NC * _NS
_G = 1


def kernel(x, emb_table):
    B, S = x.shape
    V, D = emb_table.shape
    rows = S * _G // _NW
    planes = B // _G

    @functools.partial(
        pl.kernel,
        mesh=plsc.VectorSubcoreMesh(core_axis_name="c", subcore_axis_name="s"),
        out_type=jax.ShapeDtypeStruct((B, S, D), jnp.float32),
        scratch_types=[
            pltpu.VMEM((rows, D), jnp.float32),
            pltpu.VMEM((rows, D), jnp.float32),
            pltpu.VMEM((rows, D), jnp.float32),
            pltpu.SemaphoreType.DMA,
        ],
    )
    def k(x_hbm, tab_hbm, out_hbm, xv, tv, yv, sem):
        wid = lax.axis_index("s") * _NC + lax.axis_index("c")
        group = wid // _G
        member = wid % _G
        s0 = group * rows

        @pl.when(s0 < V)
        def _():
            pltpu.sync_copy(x_hbm.at[pl.ds(s0, rows)], xv)
            pltpu.sync_copy(tab_hbm.at[pl.ds(s0, rows)], tv)
            for r in range(rows):
                for j in range(D // _NL):
                    sl = pl.ds(j * _NL, _NL)
                    yv[r, sl] = xv[r, sl] + tv[r, sl]

        @pl.when(s0 >= V)
        def _():
            nan16 = jnp.full((_NL,), jnp.nan, dtype=jnp.float32)
            for r in range(rows):
                for j in range(D // _NL):
                    yv[r, pl.ds(j * _NL, _NL)] = nan16

        b0 = member * planes
        unroll = 8

        def fire(g, carry):
            for u in range(unroll):
                pltpu.async_copy(yv, out_hbm.at[b0 + g * unroll + u,
                                                pl.ds(s0, rows)], sem)
            return carry

        lax.fori_loop(0, planes // unroll, fire, 0)
        pltpu.make_async_copy(
            out_hbm.at[pl.ds(b0, planes), pl.ds(s0, rows)],
            out_hbm.at[pl.ds(b0, planes), pl.ds(s0, rows)], sem).wait()

    return k(x, emb_table)
